# baseline (device time: 140786 ns/iter reference)
import functools

import jax
import jax.numpy as jnp
from jax import lax
from jax.experimental import pallas as pl
from jax.experimental.pallas import tpu as pltpu

NZ = 4
S = 4


def kernel(partial, gamma):
    _, m, d = partial.shape
    ch = m // NZ
    qh = ch // 4

    def body(x_ref, g_ref, out_ref, sbuf, rbuf, gath,
             z_send_sems, z_recv_sems, xy_send_sems, xy_recv_sems,
             copy_sems):
        my_x = lax.axis_index("x")
        my_y = lax.axis_index("y")
        my_z = lax.axis_index("z")
        p_me = my_x * 2 + my_y

        z_peers = [(my_x, my_y, (my_z + r) % NZ) for r in (1, 2, 3)]
        xy_offsets = [(1, 0), (0, 1), (1, 1)]
        xy_peers = [(my_x ^ dx, my_y ^ dy, my_z) for dx, dy in xy_offsets]

        copies = []
        for i, c in enumerate(
            [(my_z + r) % NZ for r in (1, 2, 3)] + [my_z]
        ):
            cp = pltpu.make_async_copy(
                x_ref.at[0, pl.ds(c * ch + p_me * qh, qh), :],
                out_ref.at[pl.ds(i * qh, qh), :],
                copy_sems.at[i],
            )
            cp.start()
            copies.append(cp)

        barrier = pltpu.get_barrier_semaphore()
        for nbr in z_peers + xy_peers:
            pl.semaphore_signal(
                barrier, inc=1, device_id=nbr,
                device_id_type=pl.DeviceIdType.MESH,
            )
        pl.semaphore_wait(barrier, 6)
        for cp in copies:
            cp.wait()

        ph = qh // S

        def z_descriptor(j, i):
            return pltpu.make_async_remote_copy(
                src_ref=sbuf.at[j, pl.ds(i * ph, ph), :],
                dst_ref=rbuf.at[j, pl.ds(i * ph, ph), :],
                send_sem=z_send_sems.at[j, i],
                recv_sem=z_recv_sems.at[j, i],
                device_id=z_peers[j],
                device_id_type=pl.DeviceIdType.MESH,
            )

        def start_z_piece(i):
            for j in range(3):
                sbuf[j, pl.ds(i * ph, ph), :] = (
                    out_ref[pl.ds(j * qh + i * ph, ph), :].astype(jnp.bfloat16)
                )
            for j in range(3):
                z_descriptor(j, i).start()

        start_z_piece(0)
        xy_rdmas = []
        for i in range(S):
            for j in range(3):
                z_descriptor(j, i).wait()
            if i + 1 < S:
                start_z_piece(i + 1)
            rows = pl.ds(3 * qh + i * ph, ph)
            qsum = out_ref[rows, :]
            for j in range(3):
                qsum = qsum + rbuf[j, pl.ds(i * ph, ph), :].astype(jnp.float32)
            gath[p_me, pl.ds(i * ph, ph), :] = qsum.astype(jnp.bfloat16)

            for j, peer in enumerate(xy_peers):
                rdma = pltpu.make_async_remote_copy(
                    src_ref=gath.at[p_me, pl.ds(i * ph, ph), :],
                    dst_ref=gath.at[p_me, pl.ds(i * ph, ph), :],
                    send_sem=xy_send_sems.at[j, i],
                    recv_sem=xy_recv_sems.at[j, i],
                    device_id=peer,
                    device_id_type=pl.DeviceIdType.MESH,
                )
                rdma.start()
                xy_rdmas.append(rdma)

        g = g_ref[0, :][None, :]
        for i in range(S):
            for rdma in xy_rdmas[3 * i : 3 * i + 3]:
                rdma.wait()
            for q in range(4):
                rows = pl.ds(q * qh + i * ph, ph)
                acc = gath[q, pl.ds(i * ph, ph), :].astype(jnp.float32)
                rms = jnp.sqrt(
                    jnp.mean(acc * acc, axis=-1, keepdims=True) + 1e-6
                )
                out_ref[rows, :] = acc / rms * g

        @functools.partial(pl.run_scoped, sem=pltpu.SemaphoreType.REGULAR)
        def _(sem):
            for nbr in z_peers + xy_peers:
                pl.semaphore_signal(
                    sem, inc=1, device_id=nbr,
                    device_id_type=pl.DeviceIdType.MESH,
                )
            pl.semaphore_wait(sem, 6)

    return pl.pallas_call(
        body,
        out_shape=jax.ShapeDtypeStruct((ch, d), jnp.float32),
        in_specs=[
            pl.BlockSpec(memory_space=pl.ANY),
            pl.BlockSpec(memory_space=pltpu.VMEM),
        ],
        out_specs=pl.BlockSpec(memory_space=pltpu.VMEM),
        scratch_shapes=[
            pltpu.VMEM((3, qh, d), jnp.bfloat16),
            pltpu.VMEM((3, qh, d), jnp.bfloat16),
            pltpu.VMEM((4, qh, d), jnp.bfloat16),
            pltpu.SemaphoreType.DMA((3, S)),
            pltpu.SemaphoreType.DMA((3, S)),
            pltpu.SemaphoreType.DMA((3, S)),
            pltpu.SemaphoreType.DMA((3, S)),
            pltpu.SemaphoreType.DMA((4,)),
        ],
        compiler_params=pltpu.CompilerParams(
            collective_id=0,
            vmem_limit_bytes=63 * 1024 * 1024,
        ),
    )(partial, gamma.reshape(1, d))
